# baseline (speedup 1.0000x reference)
import functools

import jax
import jax.numpy as jnp
from jax import lax
from jax.experimental import pallas as pl
from jax.experimental.pallas import tpu as pltpu
from jax.experimental.pallas import tpu_sc as plsc

_N = 100000
_E = 1600000
_B = 128
_NHID = 128
_NCLS = 2

_T = 2048
_G = 49
_N1 = _G * _T
_NW = 32
_EPW = 51200
_EPAD = _NW * _EPW
_ER = _EPAD // 128
_RPC = _N1 // 16
_RSTG = _RPC // 4
_CHUNK_ROWS = 16
_NCHUNK = _EPW // (_CHUNK_ROWS * 128)


def _sc_mesh():
    return plsc.VectorSubcoreMesh(
        core_axis_name="c", subcore_axis_name="s", num_cores=2, num_subcores=16
    )


def _sc_degree(dst2d):
    @functools.partial(
        pl.kernel,
        out_type=jax.ShapeDtypeStruct((2 * _N1,), jnp.float32),
        mesh=_sc_mesh(),
        scratch_types=[
            pltpu.VMEM((_CHUNK_ROWS, 128), jnp.int32),
            pltpu.VMEM((_RPC,), jnp.float32),
            pltpu.VMEM((128,), jnp.float32),
            pltpu.VMEM_SHARED((_N1,), jnp.float32),
        ],
    )
    def k(dst_hbm, deg_out, dst_buf, zeros_buf, ones_buf, deg_sh):
        c = lax.axis_index("c")
        s = lax.axis_index("s")
        gw = c * 16 + s

        def fill_z(i, _):
            zeros_buf[pl.ds(i * 16, 16)] = jnp.zeros((16,), jnp.float32)
            return 0

        lax.fori_loop(0, _RPC // 16, fill_z, 0)
        for i in range(8):
            ones_buf[pl.ds(i * 16, 16)] = jnp.ones((16,), jnp.float32)
        pltpu.sync_copy(zeros_buf, deg_sh.at[pl.ds(s * _RPC, _RPC)])
        plsc.subcore_barrier()

        base = gw * (_CHUNK_ROWS * _NCHUNK)

        def chunk(j, _):
            pltpu.sync_copy(dst_hbm.at[pl.ds(base + j * _CHUNK_ROWS, _CHUNK_ROWS)],
                            dst_buf)
            for kk in range(_CHUNK_ROWS):
                pltpu.sync_copy(ones_buf, deg_sh.at[dst_buf.at[kk]], add=True)
            return 0

        lax.fori_loop(0, _NCHUNK, chunk, 0)
        plsc.subcore_barrier()
        pltpu.sync_copy(deg_sh.at[pl.ds(s * _RPC, _RPC)], zeros_buf)
        pltpu.sync_copy(zeros_buf,
                        deg_out.at[pl.ds(c * _N1 + s * _RPC, _RPC)])

    return k(dst2d)


def _sc_scatter(src2d, dst2d, y4, zeros):
    @functools.partial(
        pl.kernel,
        out_type=jax.ShapeDtypeStruct((2 * _N1, 4), jnp.float32),
        mesh=_sc_mesh(),
        scratch_types=[
            pltpu.VMEM((_CHUNK_ROWS, 128), jnp.int32),
            pltpu.VMEM((_CHUNK_ROWS, 128), jnp.int32),
            pltpu.VMEM((_CHUNK_ROWS * 128, 4), jnp.float32),
            pltpu.VMEM_SHARED((_N1, 4), jnp.float32),
            pltpu.SemaphoreType.DMA,
        ],
        compiler_params=pltpu.CompilerParams(use_tc_tiling_on_sc=False),
    )
    def k(src_hbm, dst_hbm, y4_hbm, zeros_hbm, zp_out,
          src_buf, dst_buf, rows_buf, z_sh, sem):
        c = lax.axis_index("c")
        s = lax.axis_index("s")
        gw = c * 16 + s

        stage = rows_buf.at[pl.ds(0, _RSTG)]
        pltpu.sync_copy(zeros_hbm, stage)
        for t in range(4):
            pltpu.sync_copy(stage, z_sh.at[pl.ds(s * _RPC + t * _RSTG, _RSTG)])
        plsc.subcore_barrier()

        base = gw * (_CHUNK_ROWS * _NCHUNK)

        def chunk(j, _):
            r0 = base + j * _CHUNK_ROWS
            pltpu.sync_copy(src_hbm.at[pl.ds(r0, _CHUNK_ROWS)], src_buf)
            pltpu.sync_copy(dst_hbm.at[pl.ds(r0, _CHUNK_ROWS)], dst_buf)
            descs = [
                pltpu.async_copy(y4_hbm.at[src_buf.at[kk]],
                                 rows_buf.at[pl.ds(kk * 128, 128)], sem)
                for kk in range(_CHUNK_ROWS)
            ]
            for d in descs:
                d.wait()
            for kk in range(_CHUNK_ROWS):
                pltpu.sync_copy(rows_buf.at[pl.ds(kk * 128, 128)],
                                z_sh.at[dst_buf.at[kk]], add=True)
            return 0

        lax.fori_loop(0, _NCHUNK, chunk, 0)
        plsc.subcore_barrier()
        for t in range(4):
            pltpu.sync_copy(z_sh.at[pl.ds(s * _RPC + t * _RSTG, _RSTG)], stage)
            pltpu.sync_copy(
                stage, zp_out.at[pl.ds(c * _N1 + s * _RPC + t * _RSTG, _RSTG)])

    return k(src2d, dst2d, y4, zeros)


def _tc_prep(degp, x4):
    ra = 3136

    def body(degp_ref, x4_ref, y4_ref, dis_ref):
        deg = degp_ref[0] + degp_ref[1] + 1.0
        dis = lax.rsqrt(deg)
        dis_ref[...] = dis
        y4_ref[...] = dis * x4_ref[...]

    return pl.pallas_call(
        body,
        grid=(_N1 // ra,),
        in_specs=[
            pl.BlockSpec((2, ra, 1), lambda i: (0, i, 0)),
            pl.BlockSpec((ra, 4), lambda i: (i, 0)),
        ],
        out_specs=[
            pl.BlockSpec((ra, 4), lambda i: (i, 0)),
            pl.BlockSpec((ra, 1), lambda i: (i, 0)),
        ],
        out_shape=[
            jax.ShapeDtypeStruct((_N1, 4), jnp.float32),
            jax.ShapeDtypeStruct((_N1, 1), jnp.float32),
        ],
    )(degp, x4)


def _tc_head(zp, dis3, x43, batch3, bounds, W14, b1r, W2, b2r):
    neg_inf = float("-inf")

    def body(zp_ref, dis_ref, x4_ref, bt_ref, bounds_ref,
             w1_ref, b1_ref, w2_ref, b2_ref, out_ref, pooled):
        i = pl.program_id(0)

        @pl.when(i == 0)
        def _():
            pooled[...] = jnp.full((_B + 8, _NHID), neg_inf, jnp.float32)

        z = zp_ref[0] + zp_ref[1]
        dis = dis_ref[0]
        aggx = dis * z + (dis * dis) * x4_ref[0]
        h = jnp.dot(aggx, w1_ref[...], preferred_element_type=jnp.float32)
        h = jnp.maximum(h + b1_ref[...], 0.0)
        bt = bt_ref[0]

        bmin = bounds_ref[0, 0, 0]
        bmax = bounds_ref[0, 0, 1]

        def upd(b, _):
            m = jnp.where(bt == b, h, neg_inf)
            row = jnp.max(m, axis=0, keepdims=True)
            pooled[pl.ds(b, 1), :] = jnp.maximum(pooled[pl.ds(b, 1), :], row)
            return 0

        lax.fori_loop(bmin, bmax + 1, upd, 0)

        @pl.when(i == _G - 1)
        def _():
            p = pooled[0:_B, :]
            logits = jnp.dot(p, w2_ref[...],
                             preferred_element_type=jnp.float32) + b2_ref[...]
            mx = jnp.max(logits, axis=1, keepdims=True)
            e = jnp.exp(logits - mx)
            lse = jnp.log(jnp.sum(e, axis=1, keepdims=True)) + mx
            out_ref[...] = logits - lse

    return pl.pallas_call(
        body,
        grid=(_G,),
        in_specs=[
            pl.BlockSpec((2, _T, 4), lambda i: (0, i, 0)),
            pl.BlockSpec((1, _T, 1), lambda i: (i, 0, 0)),
            pl.BlockSpec((1, _T, 4), lambda i: (i, 0, 0)),
            pl.BlockSpec((1, _T, 1), lambda i: (i, 0, 0)),
            pl.BlockSpec((1, 1, 2), lambda i: (i, 0, 0), memory_space=pltpu.SMEM),
            pl.BlockSpec((4, _NHID), lambda i: (0, 0)),
            pl.BlockSpec((1, _NHID), lambda i: (0, 0)),
            pl.BlockSpec((_NHID, _NCLS), lambda i: (0, 0)),
            pl.BlockSpec((1, _NCLS), lambda i: (0, 0)),
        ],
        out_specs=pl.BlockSpec((_B, _NCLS), lambda i: (0, 0)),
        out_shape=jax.ShapeDtypeStruct((_B, _NCLS), jnp.float32),
        scratch_shapes=[pltpu.VMEM((_B + 8, _NHID), jnp.float32)],
    )(zp, dis3, x43, batch3, bounds, W14, b1r, W2, b2r)


def kernel(x, edge_index, batch, W1, b1, W2, b2):
    src = edge_index[0]
    dst = edge_index[1]
    pad_e = _EPAD - _E
    src2d = jnp.concatenate(
        [src, jnp.full((pad_e,), _N, jnp.int32)]).reshape(_ER, 128)
    dst2d = jnp.concatenate(
        [dst, jnp.full((pad_e,), _N, jnp.int32)]).reshape(_ER, 128)
    x4 = jnp.pad(x, ((0, _N1 - _N), (0, 4 - x.shape[1])))

    deg_flat = _sc_degree(dst2d)
    degp = deg_flat.reshape(2, _N1, 1)
    y4, dis = _tc_prep(degp, x4)

    zeros = jnp.zeros((_RSTG, 4), jnp.float32)
    zp = _sc_scatter(src2d, dst2d, y4, zeros)

    batchp = jnp.concatenate([batch, jnp.full((_N1 - _N,), _B, jnp.int32)])
    bp = batchp.reshape(_G, _T)
    bounds = jnp.stack([bp.min(axis=1), bp.max(axis=1)], axis=1)

    W14 = jnp.pad(W1, ((0, 4 - W1.shape[0]), (0, 0)))

    return _tc_head(
        zp.reshape(2, _N1, 4),
        dis.reshape(_G, _T, 1),
        x4.reshape(_G, _T, 4),
        bp.reshape(_G, _T, 1),
        bounds.reshape(_G, 1, 2),
        W14,
        b1.reshape(1, _NHID),
        W2,
        b2.reshape(1, _NCLS),
    )


# speedup vs baseline: 1.1178x; 1.1178x over previous
import functools

import jax
import jax.numpy as jnp
from jax import lax
from jax.experimental import pallas as pl
from jax.experimental.pallas import tpu as pltpu
from jax.experimental.pallas import tpu_sc as plsc

_N = 100000
_E = 1600000
_B = 128
_NHID = 128
_NCLS = 2

_T = 2048
_G = 49
_N1 = _G * _T
_NW = 32
_EPW = 51200
_EPAD = _NW * _EPW
_ER = _EPAD // 128
_RPC = _N1 // 16
_RSTG = _RPC // 4
_CHUNK_ROWS = 25
_DCH = 16
_DNC = _EPW // (_DCH * 128)
_NCHUNK = _EPW // (_CHUNK_ROWS * 128)


def _sc_mesh():
    return plsc.VectorSubcoreMesh(
        core_axis_name="c", subcore_axis_name="s", num_cores=2, num_subcores=16
    )


def _sc_degree(dst2d):
    @functools.partial(
        pl.kernel,
        out_type=jax.ShapeDtypeStruct((2 * _N1,), jnp.float32),
        mesh=_sc_mesh(),
        scratch_types=[
            pltpu.VMEM((_DCH, 128), jnp.int32),
            pltpu.VMEM((_RPC,), jnp.float32),
            pltpu.VMEM((128,), jnp.float32),
            pltpu.VMEM_SHARED((_N1,), jnp.float32),
        ],
    )
    def k(dst_hbm, deg_out, dst_buf, zeros_buf, ones_buf, deg_sh):
        c = lax.axis_index("c")
        s = lax.axis_index("s")
        gw = c * 16 + s

        def fill_z(i, _):
            zeros_buf[pl.ds(i * 16, 16)] = jnp.zeros((16,), jnp.float32)
            return 0

        lax.fori_loop(0, _RPC // 16, fill_z, 0)
        for i in range(8):
            ones_buf[pl.ds(i * 16, 16)] = jnp.ones((16,), jnp.float32)
        pltpu.sync_copy(zeros_buf, deg_sh.at[pl.ds(s * _RPC, _RPC)])
        plsc.subcore_barrier()

        base = gw * (_DCH * _DNC)

        def chunk(j, _):
            pltpu.sync_copy(dst_hbm.at[pl.ds(base + j * _DCH, _DCH)],
                            dst_buf)
            for kk in range(_DCH):
                pltpu.sync_copy(ones_buf, deg_sh.at[dst_buf.at[kk]], add=True)
            return 0

        lax.fori_loop(0, _DNC, chunk, 0)
        plsc.subcore_barrier()
        pltpu.sync_copy(deg_sh.at[pl.ds(s * _RPC, _RPC)], zeros_buf)
        pltpu.sync_copy(zeros_buf,
                        deg_out.at[pl.ds(c * _N1 + s * _RPC, _RPC)])

    return k(dst2d)


def _sc_scatter(src2d, dst2d, y8, zeros):
    ch_e = _CHUNK_ROWS * 128

    @functools.partial(
        pl.kernel,
        out_type=jax.ShapeDtypeStruct((2 * _N1, 8), jnp.float32),
        mesh=_sc_mesh(),
        scratch_types=[
            pltpu.VMEM((2, _CHUNK_ROWS, 128), jnp.int32),
            pltpu.VMEM((2, _CHUNK_ROWS, 128), jnp.int32),
            pltpu.VMEM((2, ch_e, 8), jnp.float32),
            pltpu.VMEM_SHARED((_N1, 8), jnp.float32),
            pltpu.SemaphoreType.DMA,
            pltpu.SemaphoreType.DMA,
            pltpu.SemaphoreType.DMA,
            pltpu.SemaphoreType.DMA,
        ],
        compiler_params=pltpu.CompilerParams(use_tc_tiling_on_sc=False),
    )
    def k(src_hbm, dst_hbm, y8_hbm, zeros_hbm, zp_out,
          src_buf, dst_buf, rows_buf, z_sh,
          sem_g0, sem_g1, sem_s0, sem_s1):
        c = lax.axis_index("c")
        s = lax.axis_index("s")
        gw = c * 16 + s
        sem_g = (sem_g0, sem_g1)
        sem_s = (sem_s0, sem_s1)

        stage = rows_buf.at[0, pl.ds(0, _RSTG)]
        pltpu.sync_copy(zeros_hbm, stage)
        for t in range(4):
            pltpu.sync_copy(stage, z_sh.at[pl.ds(s * _RPC + t * _RSTG, _RSTG)])
        plsc.subcore_barrier()

        base = gw * (_CHUNK_ROWS * _NCHUNK)

        def stage_idx(j, b):
            r0 = base + j * _CHUNK_ROWS
            pltpu.sync_copy(src_hbm.at[pl.ds(r0, _CHUNK_ROWS)], src_buf.at[b])
            pltpu.sync_copy(dst_hbm.at[pl.ds(r0, _CHUNK_ROWS)], dst_buf.at[b])

        def fire_gathers(b):
            for kk in range(_CHUNK_ROWS):
                pltpu.async_copy(y8_hbm.at[src_buf.at[b, kk]],
                                 rows_buf.at[b, pl.ds(kk * 128, 128)],
                                 sem_g[b])

        def fire_scatters(b):
            for kk in range(_CHUNK_ROWS):
                pltpu.async_copy(rows_buf.at[b, pl.ds(kk * 128, 128)],
                                 z_sh.at[dst_buf.at[b, kk]],
                                 sem_s[b], add=True)

        def drain_gathers(b):
            pltpu.make_async_copy(y8_hbm.at[pl.ds(0, ch_e)],
                                  rows_buf.at[b], sem_g[b]).wait()

        def drain_scatters(b):
            pltpu.make_async_copy(rows_buf.at[b],
                                  z_sh.at[pl.ds(0, ch_e)], sem_s[b]).wait()

        stage_idx(0, 0)
        fire_gathers(0)

        def pair(jj, _):
            j0 = 2 * jj

            @pl.when(jj >= 1)
            def _():
                drain_scatters(1)
            stage_idx(j0 + 1, 1)
            fire_gathers(1)
            drain_gathers(0)
            fire_scatters(0)

            @pl.when(jj + 1 < _NCHUNK // 2)
            def _():
                drain_scatters(0)
                stage_idx(j0 + 2, 0)
                fire_gathers(0)
            drain_gathers(1)
            fire_scatters(1)
            return 0

        lax.fori_loop(0, _NCHUNK // 2, pair, 0)
        drain_scatters(0)
        drain_scatters(1)
        plsc.subcore_barrier()
        for t in range(4):
            pltpu.sync_copy(z_sh.at[pl.ds(s * _RPC + t * _RSTG, _RSTG)], stage)
            pltpu.sync_copy(
                stage, zp_out.at[pl.ds(c * _N1 + s * _RPC + t * _RSTG, _RSTG)])

    return k(src2d, dst2d, y8, zeros)


def _tc_prep(degp, x8):
    ra = 3136

    def body(degp_ref, x8_ref, y8_ref, dis_ref):
        deg = degp_ref[0] + degp_ref[1] + 1.0
        dis = lax.rsqrt(deg)
        dis_ref[...] = dis
        y8_ref[...] = dis * x8_ref[...]

    return pl.pallas_call(
        body,
        grid=(_N1 // ra,),
        in_specs=[
            pl.BlockSpec((2, ra, 1), lambda i: (0, i, 0)),
            pl.BlockSpec((ra, 8), lambda i: (i, 0)),
        ],
        out_specs=[
            pl.BlockSpec((ra, 8), lambda i: (i, 0)),
            pl.BlockSpec((ra, 1), lambda i: (i, 0)),
        ],
        out_shape=[
            jax.ShapeDtypeStruct((_N1, 8), jnp.float32),
            jax.ShapeDtypeStruct((_N1, 1), jnp.float32),
        ],
    )(degp, x8)


def _tc_head(zp, dis3, x83, batch3, bounds, W18, b1r, W2, b2r):
    neg_inf = float("-inf")

    def body(zp_ref, dis_ref, x8_ref, bt_ref, bounds_ref,
             w1_ref, b1_ref, w2_ref, b2_ref, out_ref, pooled):
        i = pl.program_id(0)

        @pl.when(i == 0)
        def _():
            pooled[...] = jnp.full((_B + 8, _NHID), neg_inf, jnp.float32)

        z = zp_ref[0] + zp_ref[1]
        dis = dis_ref[0]
        aggx = dis * z + (dis * dis) * x8_ref[0]
        h = jnp.dot(aggx, w1_ref[...], preferred_element_type=jnp.float32)
        h = jnp.maximum(h + b1_ref[...], 0.0)
        bt = bt_ref[0]

        bmin = bounds_ref[0, 0, 0]
        bmax = bounds_ref[0, 0, 1]

        def upd(b, _):
            m = jnp.where(bt == b, h, neg_inf)
            row = jnp.max(m, axis=0, keepdims=True)
            pooled[pl.ds(b, 1), :] = jnp.maximum(pooled[pl.ds(b, 1), :], row)
            return 0

        lax.fori_loop(bmin, bmax + 1, upd, 0)

        @pl.when(i == _G - 1)
        def _():
            p = pooled[0:_B, :]
            logits = jnp.dot(p, w2_ref[...],
                             preferred_element_type=jnp.float32) + b2_ref[...]
            mx = jnp.max(logits, axis=1, keepdims=True)
            e = jnp.exp(logits - mx)
            lse = jnp.log(jnp.sum(e, axis=1, keepdims=True)) + mx
            out_ref[...] = logits - lse

    return pl.pallas_call(
        body,
        grid=(_G,),
        in_specs=[
            pl.BlockSpec((2, _T, 8), lambda i: (0, i, 0)),
            pl.BlockSpec((1, _T, 1), lambda i: (i, 0, 0)),
            pl.BlockSpec((1, _T, 8), lambda i: (i, 0, 0)),
            pl.BlockSpec((1, _T, 1), lambda i: (i, 0, 0)),
            pl.BlockSpec((1, 1, 2), lambda i: (i, 0, 0), memory_space=pltpu.SMEM),
            pl.BlockSpec((8, _NHID), lambda i: (0, 0)),
            pl.BlockSpec((1, _NHID), lambda i: (0, 0)),
            pl.BlockSpec((_NHID, _NCLS), lambda i: (0, 0)),
            pl.BlockSpec((1, _NCLS), lambda i: (0, 0)),
        ],
        out_specs=pl.BlockSpec((_B, _NCLS), lambda i: (0, 0)),
        out_shape=jax.ShapeDtypeStruct((_B, _NCLS), jnp.float32),
        scratch_shapes=[pltpu.VMEM((_B + 8, _NHID), jnp.float32)],
    )(zp, dis3, x83, batch3, bounds, W18, b1r, W2, b2r)


def kernel(x, edge_index, batch, W1, b1, W2, b2):
    src = edge_index[0]
    dst = edge_index[1]
    pad_e = _EPAD - _E
    src2d = jnp.concatenate(
        [src, jnp.full((pad_e,), _N, jnp.int32)]).reshape(_ER, 128)
    dst2d = jnp.concatenate(
        [dst, jnp.full((pad_e,), _N, jnp.int32)]).reshape(_ER, 128)
    x8 = jnp.pad(x, ((0, _N1 - _N), (0, 8 - x.shape[1])))

    deg_flat = _sc_degree(dst2d)
    degp = deg_flat.reshape(2, _N1, 1)
    y8, dis = _tc_prep(degp, x8)

    zeros = jnp.zeros((_RSTG, 8), jnp.float32)
    zp = _sc_scatter(src2d, dst2d, y8, zeros)

    batchp = jnp.concatenate([batch, jnp.full((_N1 - _N,), _B, jnp.int32)])
    bp = batchp.reshape(_G, _T)
    bounds = jnp.stack([bp.min(axis=1), bp.max(axis=1)], axis=1)

    W18 = jnp.pad(W1, ((0, 8 - W1.shape[0]), (0, 0)))

    return _tc_head(
        zp.reshape(2, _N1, 8),
        dis.reshape(_G, _T, 1),
        x8.reshape(_G, _T, 8),
        bp.reshape(_G, _T, 1),
        bounds.reshape(_G, 1, 2),
        W18,
        b1.reshape(1, _NHID),
        W2,
        b2.reshape(1, _NCLS),
    )


# speedup vs baseline: 1.3344x; 1.1938x over previous
import functools

import jax
import jax.numpy as jnp
from jax import lax
from jax.experimental import pallas as pl
from jax.experimental.pallas import tpu as pltpu
from jax.experimental.pallas import tpu_sc as plsc

_N = 100000
_E = 1600000
_B = 128
_NHID = 128
_NCLS = 2

_T = 2048
_G = 49
_N1 = _G * _T
_NW = 32
_EPW = 51200
_EPAD = _NW * _EPW
_ER = _EPAD // 128
_RPC = _N1 // 16
_RSTG = _RPC // 4
_CHUNK_ROWS = 10
_DCH = 16
_DNC = _EPW // (_DCH * 128)
_NCHUNK = _EPW // (_CHUNK_ROWS * 128)


def _sc_mesh():
    return plsc.VectorSubcoreMesh(
        core_axis_name="c", subcore_axis_name="s", num_cores=2, num_subcores=16
    )


def _sc_degree(dst2d):
    @functools.partial(
        pl.kernel,
        out_type=jax.ShapeDtypeStruct((2 * _N1,), jnp.float32),
        mesh=_sc_mesh(),
        scratch_types=[
            pltpu.VMEM((_DCH, 128), jnp.int32),
            pltpu.VMEM((_RPC,), jnp.float32),
            pltpu.VMEM((128,), jnp.float32),
            pltpu.VMEM_SHARED((_N1,), jnp.float32),
        ],
    )
    def k(dst_hbm, deg_out, dst_buf, zeros_buf, ones_buf, deg_sh):
        c = lax.axis_index("c")
        s = lax.axis_index("s")
        gw = c * 16 + s

        def fill_z(i, _):
            zeros_buf[pl.ds(i * 16, 16)] = jnp.zeros((16,), jnp.float32)
            return 0

        lax.fori_loop(0, _RPC // 16, fill_z, 0)
        for i in range(8):
            ones_buf[pl.ds(i * 16, 16)] = jnp.ones((16,), jnp.float32)
        pltpu.sync_copy(zeros_buf, deg_sh.at[pl.ds(s * _RPC, _RPC)])
        plsc.subcore_barrier()

        base = gw * (_DCH * _DNC)

        def chunk(j, _):
            pltpu.sync_copy(dst_hbm.at[pl.ds(base + j * _DCH, _DCH)],
                            dst_buf)
            for kk in range(_DCH):
                pltpu.sync_copy(ones_buf, deg_sh.at[dst_buf.at[kk]], add=True)
            return 0

        lax.fori_loop(0, _DNC, chunk, 0)
        plsc.subcore_barrier()
        pltpu.sync_copy(deg_sh.at[pl.ds(s * _RPC, _RPC)], zeros_buf)
        pltpu.sync_copy(zeros_buf,
                        deg_out.at[pl.ds(c * _N1 + s * _RPC, _RPC)])

    return k(dst2d)


def _sc_scatter(src2d, dst2d, y8, zeros):
    ch_e = _CHUNK_ROWS * 128

    @functools.partial(
        pl.kernel,
        out_type=jax.ShapeDtypeStruct((2 * _N1, 8), jnp.float32),
        mesh=_sc_mesh(),
        scratch_types=[
            pltpu.VMEM((2, _CHUNK_ROWS, 128), jnp.int32),
            pltpu.VMEM((2, _CHUNK_ROWS, 128), jnp.int32),
            pltpu.VMEM((2, ch_e, 8), jnp.float32),
            pltpu.VMEM_SHARED((_N1, 8), jnp.float32),
            pltpu.VMEM_SHARED((_N1, 8), jnp.float32),
            pltpu.SemaphoreType.DMA,
            pltpu.SemaphoreType.DMA,
            pltpu.SemaphoreType.DMA,
            pltpu.SemaphoreType.DMA,
        ],
        compiler_params=pltpu.CompilerParams(use_tc_tiling_on_sc=False),
    )
    def k(src_hbm, dst_hbm, y8_hbm, zeros_hbm, zp_out,
          src_buf, dst_buf, rows_buf, z_sh, y8_sh,
          sem_g0, sem_g1, sem_s0, sem_s1):
        c = lax.axis_index("c")
        s = lax.axis_index("s")
        gw = c * 16 + s
        sem_g = (sem_g0, sem_g1)
        sem_s = (sem_s0, sem_s1)

        stage = rows_buf.at[0, pl.ds(0, _RSTG)]
        pltpu.sync_copy(zeros_hbm, stage)
        for t in range(4):
            pltpu.sync_copy(stage, z_sh.at[pl.ds(s * _RPC + t * _RSTG, _RSTG)])
        for t in range(4):
            sl = pl.ds(s * _RPC + t * _RSTG, _RSTG)
            pltpu.sync_copy(y8_hbm.at[sl], stage)
            pltpu.sync_copy(stage, y8_sh.at[sl])
        plsc.subcore_barrier()

        base = gw * (_CHUNK_ROWS * _NCHUNK)

        def stage_idx(j, b):
            r0 = base + j * _CHUNK_ROWS
            pltpu.sync_copy(src_hbm.at[pl.ds(r0, _CHUNK_ROWS)], src_buf.at[b])
            pltpu.sync_copy(dst_hbm.at[pl.ds(r0, _CHUNK_ROWS)], dst_buf.at[b])

        def fire_gathers(b):
            for kk in range(_CHUNK_ROWS):
                pltpu.async_copy(y8_sh.at[src_buf.at[b, kk]],
                                 rows_buf.at[b, pl.ds(kk * 128, 128)],
                                 sem_g[b])

        def fire_scatters(b):
            for kk in range(_CHUNK_ROWS):
                pltpu.async_copy(rows_buf.at[b, pl.ds(kk * 128, 128)],
                                 z_sh.at[dst_buf.at[b, kk]],
                                 sem_s[b], add=True)

        def drain_gathers(b):
            pltpu.make_async_copy(y8_sh.at[pl.ds(0, ch_e)],
                                  rows_buf.at[b], sem_g[b]).wait()

        def drain_scatters(b):
            pltpu.make_async_copy(rows_buf.at[b],
                                  z_sh.at[pl.ds(0, ch_e)], sem_s[b]).wait()

        stage_idx(0, 0)
        fire_gathers(0)

        def pair(jj, _):
            j0 = 2 * jj

            @pl.when(jj >= 1)
            def _():
                drain_scatters(1)
            stage_idx(j0 + 1, 1)
            fire_gathers(1)
            drain_gathers(0)
            fire_scatters(0)

            @pl.when(jj + 1 < _NCHUNK // 2)
            def _():
                drain_scatters(0)
                stage_idx(j0 + 2, 0)
                fire_gathers(0)
            drain_gathers(1)
            fire_scatters(1)
            return 0

        lax.fori_loop(0, _NCHUNK // 2, pair, 0)
        drain_scatters(0)
        drain_scatters(1)
        plsc.subcore_barrier()
        for t in range(4):
            pltpu.sync_copy(z_sh.at[pl.ds(s * _RPC + t * _RSTG, _RSTG)], stage)
            pltpu.sync_copy(
                stage, zp_out.at[pl.ds(c * _N1 + s * _RPC + t * _RSTG, _RSTG)])

    return k(src2d, dst2d, y8, zeros)


def _tc_prep(degp, x8):
    ra = 3136

    def body(degp_ref, x8_ref, y8_ref, dis_ref):
        deg = degp_ref[0] + degp_ref[1] + 1.0
        dis = lax.rsqrt(deg)
        dis_ref[...] = dis
        y8_ref[...] = dis * x8_ref[...]

    return pl.pallas_call(
        body,
        grid=(_N1 // ra,),
        in_specs=[
            pl.BlockSpec((2, ra, 1), lambda i: (0, i, 0)),
            pl.BlockSpec((ra, 8), lambda i: (i, 0)),
        ],
        out_specs=[
            pl.BlockSpec((ra, 8), lambda i: (i, 0)),
            pl.BlockSpec((ra, 1), lambda i: (i, 0)),
        ],
        out_shape=[
            jax.ShapeDtypeStruct((_N1, 8), jnp.float32),
            jax.ShapeDtypeStruct((_N1, 1), jnp.float32),
        ],
    )(degp, x8)


def _tc_head(zp, dis3, x83, batch3, bounds, W18, b1r, W2, b2r):
    neg_inf = float("-inf")

    def body(zp_ref, dis_ref, x8_ref, bt_ref, bounds_ref,
             w1_ref, b1_ref, w2_ref, b2_ref, out_ref, pooled):
        i = pl.program_id(0)

        @pl.when(i == 0)
        def _():
            pooled[...] = jnp.full((_B + 8, _NHID), neg_inf, jnp.float32)

        z = zp_ref[0] + zp_ref[1]
        dis = dis_ref[0]
        aggx = dis * z + (dis * dis) * x8_ref[0]
        h = jnp.dot(aggx, w1_ref[...], preferred_element_type=jnp.float32)
        h = jnp.maximum(h + b1_ref[...], 0.0)
        bt = bt_ref[0]

        bmin = bounds_ref[0, 0, 0]
        bmax = bounds_ref[0, 0, 1]

        def upd(b, _):
            m = jnp.where(bt == b, h, neg_inf)
            row = jnp.max(m, axis=0, keepdims=True)
            pooled[pl.ds(b, 1), :] = jnp.maximum(pooled[pl.ds(b, 1), :], row)
            return 0

        lax.fori_loop(bmin, bmax + 1, upd, 0)

        @pl.when(i == _G - 1)
        def _():
            p = pooled[0:_B, :]
            logits = jnp.dot(p, w2_ref[...],
                             preferred_element_type=jnp.float32) + b2_ref[...]
            mx = jnp.max(logits, axis=1, keepdims=True)
            e = jnp.exp(logits - mx)
            lse = jnp.log(jnp.sum(e, axis=1, keepdims=True)) + mx
            out_ref[...] = logits - lse

    return pl.pallas_call(
        body,
        grid=(_G,),
        in_specs=[
            pl.BlockSpec((2, _T, 8), lambda i: (0, i, 0)),
            pl.BlockSpec((1, _T, 1), lambda i: (i, 0, 0)),
            pl.BlockSpec((1, _T, 8), lambda i: (i, 0, 0)),
            pl.BlockSpec((1, _T, 1), lambda i: (i, 0, 0)),
            pl.BlockSpec((1, 1, 2), lambda i: (i, 0, 0), memory_space=pltpu.SMEM),
            pl.BlockSpec((8, _NHID), lambda i: (0, 0)),
            pl.BlockSpec((1, _NHID), lambda i: (0, 0)),
            pl.BlockSpec((_NHID, _NCLS), lambda i: (0, 0)),
            pl.BlockSpec((1, _NCLS), lambda i: (0, 0)),
        ],
        out_specs=pl.BlockSpec((_B, _NCLS), lambda i: (0, 0)),
        out_shape=jax.ShapeDtypeStruct((_B, _NCLS), jnp.float32),
        scratch_shapes=[pltpu.VMEM((_B + 8, _NHID), jnp.float32)],
    )(zp, dis3, x83, batch3, bounds, W18, b1r, W2, b2r)


def kernel(x, edge_index, batch, W1, b1, W2, b2):
    src = edge_index[0]
    dst = edge_index[1]
    pad_e = _EPAD - _E
    src2d = jnp.concatenate(
        [src, jnp.full((pad_e,), _N, jnp.int32)]).reshape(_ER, 128)
    dst2d = jnp.concatenate(
        [dst, jnp.full((pad_e,), _N, jnp.int32)]).reshape(_ER, 128)
    x8 = jnp.pad(x, ((0, _N1 - _N), (0, 8 - x.shape[1])))

    deg_flat = _sc_degree(dst2d)
    degp = deg_flat.reshape(2, _N1, 1)
    y8, dis = _tc_prep(degp, x8)

    zeros = jnp.zeros((_RSTG, 8), jnp.float32)
    zp = _sc_scatter(src2d, dst2d, y8, zeros)

    batchp = jnp.concatenate([batch, jnp.full((_N1 - _N,), _B, jnp.int32)])
    bp = batchp.reshape(_G, _T)
    bounds = jnp.stack([bp.min(axis=1), bp.max(axis=1)], axis=1)

    W18 = jnp.pad(W1, ((0, 8 - W1.shape[0]), (0, 0)))

    return _tc_head(
        zp.reshape(2, _N1, 8),
        dis.reshape(_G, _T, 1),
        x8.reshape(_G, _T, 8),
        bp.reshape(_G, _T, 1),
        bounds.reshape(_G, 1, 2),
        W18,
        b1.reshape(1, _NHID),
        W2,
        b2.reshape(1, _NCLS),
    )


# speedup vs baseline: 1.3414x; 1.0052x over previous
import functools

import jax
import jax.numpy as jnp
from jax import lax
from jax.experimental import pallas as pl
from jax.experimental.pallas import tpu as pltpu
from jax.experimental.pallas import tpu_sc as plsc

_N = 100000
_E = 1600000
_B = 128
_NHID = 128
_NCLS = 2

_T = 2048
_G = 49
_N1 = _G * _T
_NW = 32
_EPW = 51200
_EPAD = _NW * _EPW
_ER = _EPAD // 128
_RPC = _N1 // 16
_RSTG = _RPC // 4
_CHUNK_ROWS = 10
_DCH = 16
_DNC = _EPW // (_DCH * 128)
_NCHUNK = _EPW // (_CHUNK_ROWS * 128)


def _sc_mesh():
    return plsc.VectorSubcoreMesh(
        core_axis_name="c", subcore_axis_name="s", num_cores=2, num_subcores=16
    )


def _sc_degree(dst2d):
    @functools.partial(
        pl.kernel,
        out_type=jax.ShapeDtypeStruct((2 * _N1,), jnp.float32),
        mesh=_sc_mesh(),
        scratch_types=[
            pltpu.VMEM((_DCH, 128), jnp.int32),
            pltpu.VMEM((_RPC,), jnp.float32),
            pltpu.VMEM((128,), jnp.float32),
            pltpu.VMEM_SHARED((_N1,), jnp.float32),
            pltpu.SemaphoreType.DMA,
        ],
    )
    def k(dst_hbm, deg_out, dst_buf, zeros_buf, ones_buf, deg_sh, sem):
        c = lax.axis_index("c")
        s = lax.axis_index("s")
        gw = c * 16 + s

        def fill_z(i, _):
            zeros_buf[pl.ds(i * 16, 16)] = jnp.zeros((16,), jnp.float32)
            return 0

        lax.fori_loop(0, _RPC // 16, fill_z, 0)
        for i in range(8):
            ones_buf[pl.ds(i * 16, 16)] = jnp.ones((16,), jnp.float32)
        pltpu.sync_copy(zeros_buf, deg_sh.at[pl.ds(s * _RPC, _RPC)])
        plsc.subcore_barrier()

        base = gw * (_DCH * _DNC)

        def chunk(j, _):
            pltpu.sync_copy(dst_hbm.at[pl.ds(base + j * _DCH, _DCH)],
                            dst_buf)
            for kk in range(_DCH):
                pltpu.async_copy(ones_buf, deg_sh.at[dst_buf.at[kk]], sem,
                                 add=True)
            pltpu.make_async_copy(zeros_buf.at[pl.ds(0, _DCH * 128)],
                                  deg_sh.at[pl.ds(0, _DCH * 128)], sem).wait()
            return 0

        lax.fori_loop(0, _DNC, chunk, 0)
        plsc.subcore_barrier()
        pltpu.sync_copy(deg_sh.at[pl.ds(s * _RPC, _RPC)], zeros_buf)
        pltpu.sync_copy(zeros_buf,
                        deg_out.at[pl.ds(c * _N1 + s * _RPC, _RPC)])

    return k(dst2d)


def _sc_scatter(src2d, dst2d, y8, zeros):
    ch_e = _CHUNK_ROWS * 128

    @functools.partial(
        pl.kernel,
        out_type=jax.ShapeDtypeStruct((2 * _N1, 8), jnp.float32),
        mesh=_sc_mesh(),
        scratch_types=[
            pltpu.VMEM((2, _CHUNK_ROWS, 128), jnp.int32),
            pltpu.VMEM((2, _CHUNK_ROWS, 128), jnp.int32),
            pltpu.VMEM((2, ch_e, 8), jnp.float32),
            pltpu.VMEM_SHARED((_N1, 8), jnp.float32),
            pltpu.VMEM_SHARED((_N1, 8), jnp.float32),
            pltpu.SemaphoreType.DMA,
            pltpu.SemaphoreType.DMA,
            pltpu.SemaphoreType.DMA,
            pltpu.SemaphoreType.DMA,
        ],
        compiler_params=pltpu.CompilerParams(use_tc_tiling_on_sc=False),
    )
    def k(src_hbm, dst_hbm, y8_hbm, zeros_hbm, zp_out,
          src_buf, dst_buf, rows_buf, z_sh, y8_sh,
          sem_g0, sem_g1, sem_s0, sem_s1):
        c = lax.axis_index("c")
        s = lax.axis_index("s")
        gw = c * 16 + s
        sem_g = (sem_g0, sem_g1)
        sem_s = (sem_s0, sem_s1)

        stage = rows_buf.at[0, pl.ds(0, _RSTG)]
        pltpu.sync_copy(zeros_hbm, stage)
        for t in range(4):
            pltpu.sync_copy(stage, z_sh.at[pl.ds(s * _RPC + t * _RSTG, _RSTG)])
        for t in range(4):
            sl = pl.ds(s * _RPC + t * _RSTG, _RSTG)
            pltpu.sync_copy(y8_hbm.at[sl], stage)
            pltpu.sync_copy(stage, y8_sh.at[sl])
        plsc.subcore_barrier()

        base = gw * (_CHUNK_ROWS * _NCHUNK)

        def stage_idx(j, b):
            r0 = base + j * _CHUNK_ROWS
            pltpu.sync_copy(src_hbm.at[pl.ds(r0, _CHUNK_ROWS)], src_buf.at[b])
            pltpu.sync_copy(dst_hbm.at[pl.ds(r0, _CHUNK_ROWS)], dst_buf.at[b])

        def fire_gathers(b):
            for kk in range(_CHUNK_ROWS):
                pltpu.async_copy(y8_sh.at[src_buf.at[b, kk]],
                                 rows_buf.at[b, pl.ds(kk * 128, 128)],
                                 sem_g[b])

        def fire_scatters(b):
            for kk in range(_CHUNK_ROWS):
                pltpu.async_copy(rows_buf.at[b, pl.ds(kk * 128, 128)],
                                 z_sh.at[dst_buf.at[b, kk]],
                                 sem_s[b], add=True)

        def drain_gathers(b):
            pltpu.make_async_copy(y8_sh.at[pl.ds(0, ch_e)],
                                  rows_buf.at[b], sem_g[b]).wait()

        def drain_scatters(b):
            pltpu.make_async_copy(rows_buf.at[b],
                                  z_sh.at[pl.ds(0, ch_e)], sem_s[b]).wait()

        stage_idx(0, 0)
        fire_gathers(0)

        def pair(jj, _):
            j0 = 2 * jj

            @pl.when(jj >= 1)
            def _():
                drain_scatters(1)
            stage_idx(j0 + 1, 1)
            fire_gathers(1)
            drain_gathers(0)
            fire_scatters(0)

            @pl.when(jj + 1 < _NCHUNK // 2)
            def _():
                drain_scatters(0)
                stage_idx(j0 + 2, 0)
                fire_gathers(0)
            drain_gathers(1)
            fire_scatters(1)
            return 0

        lax.fori_loop(0, _NCHUNK // 2, pair, 0)
        drain_scatters(0)
        drain_scatters(1)
        plsc.subcore_barrier()
        for t in range(4):
            pltpu.sync_copy(z_sh.at[pl.ds(s * _RPC + t * _RSTG, _RSTG)], stage)
            pltpu.sync_copy(
                stage, zp_out.at[pl.ds(c * _N1 + s * _RPC + t * _RSTG, _RSTG)])

    return k(src2d, dst2d, y8, zeros)


def _tc_prep(degp, x8):
    ra = 3136

    def body(degp_ref, x8_ref, y8_ref, dis_ref):
        deg = degp_ref[0] + degp_ref[1] + 1.0
        dis = lax.rsqrt(deg)
        dis_ref[...] = dis
        y8_ref[...] = dis * x8_ref[...]

    return pl.pallas_call(
        body,
        grid=(_N1 // ra,),
        in_specs=[
            pl.BlockSpec((2, ra, 1), lambda i: (0, i, 0)),
            pl.BlockSpec((ra, 8), lambda i: (i, 0)),
        ],
        out_specs=[
            pl.BlockSpec((ra, 8), lambda i: (i, 0)),
            pl.BlockSpec((ra, 1), lambda i: (i, 0)),
        ],
        out_shape=[
            jax.ShapeDtypeStruct((_N1, 8), jnp.float32),
            jax.ShapeDtypeStruct((_N1, 1), jnp.float32),
        ],
    )(degp, x8)


def _tc_head(zp, dis3, x83, batch3, bounds, W18, b1r, W2, b2r):
    neg_inf = float("-inf")

    def body(zp_ref, dis_ref, x8_ref, bt_ref, bounds_ref,
             w1_ref, b1_ref, w2_ref, b2_ref, out_ref, pooled):
        i = pl.program_id(0)

        @pl.when(i == 0)
        def _():
            pooled[...] = jnp.full((_B + 8, _NHID), neg_inf, jnp.float32)

        z = zp_ref[0] + zp_ref[1]
        dis = dis_ref[0]
        aggx = dis * z + (dis * dis) * x8_ref[0]
        h = jnp.dot(aggx, w1_ref[...], preferred_element_type=jnp.float32)
        h = jnp.maximum(h + b1_ref[...], 0.0)
        bt = bt_ref[0]

        bmin = bounds_ref[0, 0, 0]
        bmax = bounds_ref[0, 0, 1]

        def upd(b, _):
            m = jnp.where(bt == b, h, neg_inf)
            row = jnp.max(m, axis=0, keepdims=True)
            pooled[pl.ds(b, 1), :] = jnp.maximum(pooled[pl.ds(b, 1), :], row)
            return 0

        lax.fori_loop(bmin, bmax + 1, upd, 0)

        @pl.when(i == _G - 1)
        def _():
            p = pooled[0:_B, :]
            logits = jnp.dot(p, w2_ref[...],
                             preferred_element_type=jnp.float32) + b2_ref[...]
            mx = jnp.max(logits, axis=1, keepdims=True)
            e = jnp.exp(logits - mx)
            lse = jnp.log(jnp.sum(e, axis=1, keepdims=True)) + mx
            out_ref[...] = logits - lse

    return pl.pallas_call(
        body,
        grid=(_G,),
        in_specs=[
            pl.BlockSpec((2, _T, 8), lambda i: (0, i, 0)),
            pl.BlockSpec((1, _T, 1), lambda i: (i, 0, 0)),
            pl.BlockSpec((1, _T, 8), lambda i: (i, 0, 0)),
            pl.BlockSpec((1, _T, 1), lambda i: (i, 0, 0)),
            pl.BlockSpec((1, 1, 2), lambda i: (i, 0, 0), memory_space=pltpu.SMEM),
            pl.BlockSpec((8, _NHID), lambda i: (0, 0)),
            pl.BlockSpec((1, _NHID), lambda i: (0, 0)),
            pl.BlockSpec((_NHID, _NCLS), lambda i: (0, 0)),
            pl.BlockSpec((1, _NCLS), lambda i: (0, 0)),
        ],
        out_specs=pl.BlockSpec((_B, _NCLS), lambda i: (0, 0)),
        out_shape=jax.ShapeDtypeStruct((_B, _NCLS), jnp.float32),
        scratch_shapes=[pltpu.VMEM((_B + 8, _NHID), jnp.float32)],
    )(zp, dis3, x83, batch3, bounds, W18, b1r, W2, b2r)


def kernel(x, edge_index, batch, W1, b1, W2, b2):
    src = edge_index[0]
    dst = edge_index[1]
    pad_e = _EPAD - _E
    src2d = jnp.concatenate(
        [src, jnp.full((pad_e,), _N, jnp.int32)]).reshape(_ER, 128)
    dst2d = jnp.concatenate(
        [dst, jnp.full((pad_e,), _N, jnp.int32)]).reshape(_ER, 128)
    x8 = jnp.pad(x, ((0, _N1 - _N), (0, 8 - x.shape[1])))

    deg_flat = _sc_degree(dst2d)
    degp = deg_flat.reshape(2, _N1, 1)
    y8, dis = _tc_prep(degp, x8)

    zeros = jnp.zeros((_RSTG, 8), jnp.float32)
    zp = _sc_scatter(src2d, dst2d, y8, zeros)

    batchp = jnp.concatenate([batch, jnp.full((_N1 - _N,), _B, jnp.int32)])
    bp = batchp.reshape(_G, _T)
    bounds = jnp.stack([bp.min(axis=1), bp.max(axis=1)], axis=1)

    W18 = jnp.pad(W1, ((0, 8 - W1.shape[0]), (0, 0)))

    return _tc_head(
        zp.reshape(2, _N1, 8),
        dis.reshape(_G, _T, 1),
        x8.reshape(_G, _T, 8),
        bp.reshape(_G, _T, 1),
        bounds.reshape(_G, 1, 2),
        W18,
        b1.reshape(1, _NHID),
        W2,
        b2.reshape(1, _NCLS),
    )
